# baseline (speedup 1.0000x reference)
import functools

import jax
import jax.numpy as jnp
from jax import lax
from jax.experimental import pallas as pl
from jax.experimental.pallas import tpu as pltpu
from jax.experimental.pallas import tpu_sc as plsc

VOCAB = 100000
DIM = 64
BATCH = 16384

_NC = 2
_NS = 16
_NW = _NC * _NS
_BPW = BATCH // _NW
_CH = 128
_NCH = _BPW // _CH
_L = 16

_ROWS_BLK = 4096


def _rowsum_body(x_ref, o_ref):
    o_ref[...] = jnp.sum(x_ref[...], axis=1)


_rowsum = pl.pallas_call(
    _rowsum_body,
    grid=(pl.cdiv(VOCAB, _ROWS_BLK),),
    in_specs=[pl.BlockSpec((_ROWS_BLK, DIM), lambda i: (i, 0))],
    out_specs=pl.BlockSpec((_ROWS_BLK,), lambda i: (i,)),
    out_shape=jax.ShapeDtypeStruct((VOCAB,), jnp.float32),
)

_mesh = plsc.VectorSubcoreMesh(core_axis_name="c", subcore_axis_name="s")


@functools.partial(
    pl.kernel,
    mesh=_mesh,
    out_type=jax.ShapeDtypeStruct((BATCH,), jnp.float32),
    compiler_params=pltpu.CompilerParams(
        needs_layout_passes=False, use_tc_tiling_on_sc=False),
    scratch_types=[
        pltpu.VMEM((2 * _BPW,), jnp.int32),
        pltpu.VMEM((_NCH, _CH), jnp.int32),
        pltpu.VMEM((_BPW,), jnp.float32),
        pltpu.SemaphoreType.DMA,
        pltpu.SemaphoreType.DMA,
        pltpu.SemaphoreType.DMA,
        pltpu.SemaphoreType.DMA,
    ],
)
def _gather(sums_hbm, ind_hbm, out_hbm, pairs_v, idx_v, out_v, s0, s1, s2, s3):
    wid = lax.axis_index("s") * _NC + lax.axis_index("c")
    lanes = lax.broadcasted_iota(jnp.int32, (_L,), 0)
    pltpu.sync_copy(ind_hbm.at[pl.ds(wid * 2 * _BPW, 2 * _BPW)], pairs_v)
    for g in range(_BPW // _L):
        v = plsc.load_gather(pairs_v, [2 * (g * _L + lanes) + 1])
        idx_v[g // (_CH // _L), pl.ds((g % (_CH // _L)) * _L, _L)] = v
    sems = (s0, s1, s2, s3)
    copies = [
        pltpu.async_copy(sums_hbm.at[idx_v.at[j]],
                         out_v.at[pl.ds(j * _CH, _CH)], sems[j])
        for j in range(_NCH)
    ]
    for c in copies:
        c.wait()
    pltpu.sync_copy(out_v, out_hbm.at[pl.ds(wid * _BPW, _BPW)])


def kernel(train_table, indices):
    sums = _rowsum(train_table)
    return _gather(sums, indices.reshape(-1).astype(jnp.int32))


# speedup vs baseline: 2.2830x; 2.2830x over previous
import functools

import jax
import jax.numpy as jnp
from jax import lax
from jax.experimental import pallas as pl
from jax.experimental.pallas import tpu as pltpu
from jax.experimental.pallas import tpu_sc as plsc

VOCAB = 100000
DIM = 64
BATCH = 16384

_NC = 2
_NS = 16
_NW = _NC * _NS
_BPW = BATCH // _NW
_CH = 128
_NCH = _BPW // _CH
_L = 16

_COLS_BLK = 4096


def _rowsum_body(xt_ref, o_ref):
    o_ref[...] = jnp.sum(xt_ref[...], axis=0)


_rowsum = pl.pallas_call(
    _rowsum_body,
    grid=(pl.cdiv(VOCAB, _COLS_BLK),),
    in_specs=[pl.BlockSpec((DIM, _COLS_BLK), lambda i: (0, i))],
    out_specs=pl.BlockSpec((_COLS_BLK,), lambda i: (i,)),
    out_shape=jax.ShapeDtypeStruct((VOCAB,), jnp.float32),
)

_mesh = plsc.VectorSubcoreMesh(core_axis_name="c", subcore_axis_name="s")


@functools.partial(
    pl.kernel,
    mesh=_mesh,
    out_type=jax.ShapeDtypeStruct((BATCH,), jnp.float32),
    compiler_params=pltpu.CompilerParams(
        needs_layout_passes=False, use_tc_tiling_on_sc=False),
    scratch_types=[
        pltpu.VMEM((2 * _BPW,), jnp.int32),
        pltpu.VMEM((_NCH, _CH), jnp.int32),
        pltpu.VMEM((_BPW,), jnp.float32),
        pltpu.SemaphoreType.DMA,
        pltpu.SemaphoreType.DMA,
        pltpu.SemaphoreType.DMA,
        pltpu.SemaphoreType.DMA,
    ],
)
def _gather(sums_hbm, ind_hbm, out_hbm, pairs_v, idx_v, out_v, s0, s1, s2, s3):
    wid = lax.axis_index("s") * _NC + lax.axis_index("c")
    lanes = lax.broadcasted_iota(jnp.int32, (_L,), 0)
    pltpu.sync_copy(ind_hbm.at[pl.ds(wid * 2 * _BPW, 2 * _BPW)], pairs_v)
    for g in range(_BPW // _L):
        v = plsc.load_gather(pairs_v, [2 * (g * _L + lanes) + 1])
        idx_v[g // (_CH // _L), pl.ds((g % (_CH // _L)) * _L, _L)] = v
    sems = (s0, s1, s2, s3)
    copies = [
        pltpu.async_copy(sums_hbm.at[idx_v.at[j]],
                         out_v.at[pl.ds(j * _CH, _CH)], sems[j])
        for j in range(_NCH)
    ]
    for c in copies:
        c.wait()
    pltpu.sync_copy(out_v, out_hbm.at[pl.ds(wid * _BPW, _BPW)])


def kernel(train_table, indices):
    sums = _rowsum(train_table.T)
    return _gather(sums, indices.reshape(-1).astype(jnp.int32))


# speedup vs baseline: 2.9869x; 1.3083x over previous
import functools

import jax
import jax.numpy as jnp
from jax import lax
from jax.experimental import pallas as pl
from jax.experimental.pallas import tpu as pltpu
from jax.experimental.pallas import tpu_sc as plsc

VOCAB = 100000
DIM = 64
BATCH = 16384

_NC = 2
_NS = 16
_NW = _NC * _NS
_BPW = BATCH // _NW
_CH = 128
_NCH = _BPW // _CH
_L = 16

_COLS_BLK = 4096


def _rowsum_body(xt_ref, o_ref):
    o_ref[...] = jnp.sum(xt_ref[...], axis=0)


_rowsum = pl.pallas_call(
    _rowsum_body,
    grid=(pl.cdiv(VOCAB, _COLS_BLK),),
    in_specs=[pl.BlockSpec((DIM, _COLS_BLK), lambda i: (0, i))],
    out_specs=pl.BlockSpec((_COLS_BLK,), lambda i: (i,)),
    out_shape=jax.ShapeDtypeStruct((VOCAB,), jnp.float32),
)

_mesh = plsc.VectorSubcoreMesh(core_axis_name="c", subcore_axis_name="s")


@functools.partial(
    pl.kernel,
    mesh=_mesh,
    out_type=jax.ShapeDtypeStruct((BATCH,), jnp.float32),
    compiler_params=pltpu.CompilerParams(
        needs_layout_passes=False, use_tc_tiling_on_sc=False),
    scratch_types=[
        pltpu.VMEM((_NCH, _CH), jnp.int32),
        pltpu.VMEM((_BPW,), jnp.float32),
        pltpu.SemaphoreType.DMA,
        pltpu.SemaphoreType.DMA,
        pltpu.SemaphoreType.DMA,
        pltpu.SemaphoreType.DMA,
    ],
)
def _gather(sums_hbm, ind_hbm, out_hbm, idx_v, out_v, s0, s1, s2, s3):
    wid = lax.axis_index("s") * _NC + lax.axis_index("c")
    for j in range(_NCH):
        blk = wid * _NCH + j
        pltpu.sync_copy(ind_hbm.at[pl.ds(blk * 2 * _CH + _CH, _CH)],
                        idx_v.at[j])
    sems = (s0, s1, s2, s3)
    copies = [
        pltpu.async_copy(sums_hbm.at[idx_v.at[j]],
                         out_v.at[pl.ds(j * _CH, _CH)], sems[j])
        for j in range(_NCH)
    ]
    for c in copies:
        c.wait()
    pltpu.sync_copy(out_v, out_hbm.at[pl.ds(wid * _BPW, _BPW)])


def kernel(train_table, indices):
    sums = _rowsum(train_table.T)
    ind = indices.astype(jnp.int32)
    ind_flat = jnp.swapaxes(ind.reshape(BATCH // _CH, _CH, 2), 1, 2).reshape(-1)
    return _gather(sums, ind_flat)


# speedup vs baseline: 3.5024x; 1.1726x over previous
import functools

import jax
import jax.numpy as jnp
from jax import lax
from jax.experimental import pallas as pl
from jax.experimental.pallas import tpu as pltpu
from jax.experimental.pallas import tpu_sc as plsc

VOCAB = 100000
DIM = 64
BATCH = 16384

_NC = 2
_NS = 16
_NW = _NC * _NS
_BPW = BATCH // _NW
_CH = 128
_NCH = _BPW // _CH
_L = 16

_COLS_BLK = 8192


def _rowsum_body(xt_ref, o_ref):
    o_ref[...] = jnp.sum(xt_ref[...], axis=0)


_rowsum = pl.pallas_call(
    _rowsum_body,
    grid=(pl.cdiv(VOCAB, _COLS_BLK),),
    in_specs=[pl.BlockSpec((DIM, _COLS_BLK), lambda i: (0, i))],
    out_specs=pl.BlockSpec((_COLS_BLK,), lambda i: (i,)),
    out_shape=jax.ShapeDtypeStruct((VOCAB,), jnp.float32),
)

_mesh = plsc.VectorSubcoreMesh(core_axis_name="c", subcore_axis_name="s")


@functools.partial(
    pl.kernel,
    mesh=_mesh,
    out_type=jax.ShapeDtypeStruct((BATCH,), jnp.float32),
    compiler_params=pltpu.CompilerParams(
        needs_layout_passes=False, use_tc_tiling_on_sc=False),
    scratch_types=[
        pltpu.VMEM((_NCH, _CH), jnp.int32),
        pltpu.VMEM((_BPW,), jnp.float32),
        pltpu.SemaphoreType.DMA,
        pltpu.SemaphoreType.DMA,
        pltpu.SemaphoreType.DMA,
        pltpu.SemaphoreType.DMA,
    ],
)
def _gather(sums_hbm, ind_hbm, out_hbm, idx_v, out_v, s0, s1, s2, s3):
    wid = lax.axis_index("s") * _NC + lax.axis_index("c")
    for j in range(_NCH):
        blk = wid * _NCH + j
        pltpu.sync_copy(ind_hbm.at[pl.ds(blk * 2 * _CH + _CH, _CH)],
                        idx_v.at[j])
    sems = (s0, s1, s2, s3)
    copies = [
        pltpu.async_copy(sums_hbm.at[idx_v.at[j]],
                         out_v.at[pl.ds(j * _CH, _CH)], sems[j])
        for j in range(_NCH)
    ]
    for c in copies:
        c.wait()
    pltpu.sync_copy(out_v, out_hbm.at[pl.ds(wid * _BPW, _BPW)])


def kernel(train_table, indices):
    sums = _rowsum(train_table.T)
    ind = indices.astype(jnp.int32)
    ind_flat = jnp.swapaxes(ind.reshape(BATCH // _CH, _CH, 2), 1, 2).reshape(-1)
    return _gather(sums, ind_flat)


# speedup vs baseline: 3.8433x; 1.0973x over previous
import functools

import jax
import jax.numpy as jnp
from jax import lax
from jax.experimental import pallas as pl
from jax.experimental.pallas import tpu as pltpu
from jax.experimental.pallas import tpu_sc as plsc

VOCAB = 100000
DIM = 64
BATCH = 16384

_NC = 2
_NS = 16
_NW = _NC * _NS
_BPW = BATCH // _NW
_CH = 128
_NCH = _BPW // _CH
_L = 16

_COLS_BLK = 16384


def _rowsum_body(xt_ref, o_ref):
    o_ref[...] = jnp.sum(xt_ref[...], axis=0)


_rowsum = pl.pallas_call(
    _rowsum_body,
    grid=(pl.cdiv(VOCAB, _COLS_BLK),),
    in_specs=[pl.BlockSpec((DIM, _COLS_BLK), lambda i: (0, i))],
    out_specs=pl.BlockSpec((_COLS_BLK,), lambda i: (i,)),
    out_shape=jax.ShapeDtypeStruct((VOCAB,), jnp.float32),
)

_mesh = plsc.VectorSubcoreMesh(core_axis_name="c", subcore_axis_name="s")


@functools.partial(
    pl.kernel,
    mesh=_mesh,
    out_type=jax.ShapeDtypeStruct((BATCH,), jnp.float32),
    compiler_params=pltpu.CompilerParams(
        needs_layout_passes=False, use_tc_tiling_on_sc=False),
    scratch_types=[
        pltpu.VMEM((_NCH, _CH), jnp.int32),
        pltpu.VMEM((_BPW,), jnp.float32),
        pltpu.SemaphoreType.DMA,
        pltpu.SemaphoreType.DMA,
        pltpu.SemaphoreType.DMA,
        pltpu.SemaphoreType.DMA,
    ],
)
def _gather(sums_hbm, ind_hbm, out_hbm, idx_v, out_v, s0, s1, s2, s3):
    wid = lax.axis_index("s") * _NC + lax.axis_index("c")
    for j in range(_NCH):
        blk = wid * _NCH + j
        pltpu.sync_copy(ind_hbm.at[pl.ds(blk * 2 * _CH + _CH, _CH)],
                        idx_v.at[j])
    sems = (s0, s1, s2, s3)
    copies = [
        pltpu.async_copy(sums_hbm.at[idx_v.at[j]],
                         out_v.at[pl.ds(j * _CH, _CH)], sems[j])
        for j in range(_NCH)
    ]
    for c in copies:
        c.wait()
    pltpu.sync_copy(out_v, out_hbm.at[pl.ds(wid * _BPW, _BPW)])


def kernel(train_table, indices):
    sums = _rowsum(train_table.T)
    ind = indices.astype(jnp.int32)
    ind_flat = jnp.swapaxes(ind.reshape(BATCH // _CH, _CH, 2), 1, 2).reshape(-1)
    return _gather(sums, ind_flat)
